# Initial kernel scaffold; baseline (speedup 1.0000x reference)
import jax
import jax.numpy as jnp
from jax.experimental import pallas as pl


def kernel(x_recipe, x_user, edge_ur, edge_ru, emb_user, W_in, b_in, Wl0_ur, bl0_ur, Wr0_ur, Wl0_ru, bl0_ru, Wr0_ru, Wl1_ur, bl1_ur, Wr1_ur, Wl1_ru, bl1_ru, Wr1_ru):
    raise NotImplementedError("write your pallas kernel here")


# speedup vs baseline: 3.9929x; 3.9929x over previous
import functools

import jax
import jax.numpy as jnp
from jax import lax
from jax.experimental import pallas as pl
from jax.experimental.pallas import tpu as pltpu
from jax.experimental.pallas import tpu_sc as plsc

N = 10000
H = 512
NP = 4
PW = 128
E = 160000
EPAD = 163840
NPAD = 10240
ROWS_PER_SUB = NPAD // 16
IDXROWS = EPAD // 128
IDXROWS_PER_SUB = IDXROWS // 16

_MESH = plsc.VectorSubcoreMesh(core_axis_name="c", subcore_axis_name="s")


def _emb_gather_body(emb, xu_idx, out, idx_v, rows_v):
    c = lax.axis_index("c")
    s = lax.axis_index("s")
    w = s * 2 + c
    for k in range(3):
        b = w + 32 * k

        @pl.when(b < IDXROWS_PER_SUB)
        def _():
            pltpu.sync_copy(xu_idx.at[pl.ds(b, 1)], idx_v)
            pltpu.sync_copy(emb.at[idx_v.at[0]], rows_v)
            off = pl.multiple_of(b * 128, 128)
            for kk in range(NP):
                pltpu.sync_copy(rows_v.at[:, pl.ds(kk * PW, PW)],
                                out.at[kk].at[pl.ds(off, 128), :])


def _emb_gather(emb, xu_idx):
    fn = pl.kernel(
        _emb_gather_body,
        out_type=jax.ShapeDtypeStruct((NP, NPAD, PW), jnp.float32),
        mesh=_MESH,
        scratch_types=[
            pltpu.VMEM((1, 128), jnp.int32),
            pltpu.VMEM((128, H), jnp.float32),
        ],
    )
    return fn(emb, xu_idx)


def _invcnt_body(dst_ur, dst_ru, zeros, ones, out, acc, didx_v, ones_v):
    c = lax.axis_index("c")
    s = lax.axis_index("s")
    base = pl.multiple_of(s * ROWS_PER_SUB, 8)
    pltpu.sync_copy(zeros, acc.at[pl.ds(base, ROWS_PER_SUB)])
    pltpu.sync_copy(ones, ones_v)
    plsc.subcore_barrier()

    def count(dst_ref):
        def blk(i, _):
            r0 = pl.multiple_of(s * IDXROWS_PER_SUB + i * 8, 8)
            pltpu.sync_copy(dst_ref.at[pl.ds(r0, 8)], didx_v)
            for j in range(8):
                pltpu.sync_copy(ones_v, acc.at[didx_v.at[j]], add=True)
            return 0

        lax.fori_loop(0, IDXROWS_PER_SUB // 8, blk, 0)

    @pl.when(c == 0)
    def _():
        count(dst_ur)

    @pl.when(c == 1)
    def _():
        count(dst_ru)

    plsc.subcore_barrier()
    off = pl.multiple_of(c * NPAD + s * ROWS_PER_SUB, 8)
    pltpu.sync_copy(acc.at[pl.ds(base, ROWS_PER_SUB)],
                    out.at[pl.ds(off, ROWS_PER_SUB), :])


def _inv_counts(dst_ur, dst_ru, zeros, ones):
    fn = pl.kernel(
        _invcnt_body,
        out_type=jax.ShapeDtypeStruct((2 * NPAD, PW), jnp.float32),
        mesh=_MESH,
        scratch_types=[
            pltpu.VMEM_SHARED((NPAD, PW), jnp.float32),
            pltpu.VMEM((8, 128), jnp.int32),
            pltpu.VMEM((128, PW), jnp.float32),
        ],
    )
    return fn(dst_ur, dst_ru, zeros, ones)


def _segsum_body(xsrc, src_idx, dst_idx, zeros, out, acc, sidx_v, didx_v,
                 rows_v):
    c = lax.axis_index("c")
    s = lax.axis_index("s")
    base = pl.multiple_of(s * ROWS_PER_SUB, 8)
    for p in range(NP):

        @pl.when((p // 2) == c)
        def _():
            pltpu.sync_copy(zeros, acc.at[pl.ds(base, ROWS_PER_SUB)])
            plsc.subcore_barrier()

            def blk(i, _):
                r0 = pl.multiple_of(s * IDXROWS_PER_SUB + i * 8, 8)
                pltpu.sync_copy(src_idx.at[pl.ds(r0, 8)], sidx_v)
                pltpu.sync_copy(dst_idx.at[pl.ds(r0, 8)], didx_v)
                for j in range(8):
                    pltpu.sync_copy(xsrc.at[p].at[sidx_v.at[j]], rows_v)
                    pltpu.sync_copy(rows_v, acc.at[didx_v.at[j]], add=True)
                return 0

            lax.fori_loop(0, IDXROWS_PER_SUB // 8, blk, 0)
            plsc.subcore_barrier()
            pltpu.sync_copy(acc.at[pl.ds(base, ROWS_PER_SUB)],
                            out.at[p].at[pl.ds(base, ROWS_PER_SUB), :])


def _segsum(xsrc_pm, src_idx, dst_idx, zeros):
    fn = pl.kernel(
        _segsum_body,
        out_type=jax.ShapeDtypeStruct((NP, NPAD, PW), jnp.float32),
        mesh=_MESH,
        scratch_types=[
            pltpu.VMEM_SHARED((NPAD, PW), jnp.float32),
            pltpu.VMEM((8, 128), jnp.int32),
            pltpu.VMEM((8, 128), jnp.int32),
            pltpu.VMEM((128, PW), jnp.float32),
        ],
    )
    return fn(xsrc_pm, src_idx, dst_idx, zeros)


M_BLK = 400


def _sage_body(panel_out, s_ref, xd_ref, ic_ref, wl_ref, wr_ref, bl_ref,
               o_ref):
    ic = 1.0 / jnp.maximum(ic_ref[...], 1.0)
    acc = jnp.broadcast_to(bl_ref[...], (M_BLK, H)).astype(jnp.float32)
    for k in range(NP):
        wl_k = wl_ref[pl.ds(k * PW, PW), :]
        wr_k = wr_ref[pl.ds(k * PW, PW), :]
        acc = acc + jnp.dot(s_ref[k] * ic, wl_k,
                            preferred_element_type=jnp.float32)
        acc = acc + jnp.dot(xd_ref[k], wr_k,
                            preferred_element_type=jnp.float32)
    acc = jnp.maximum(acc, 0.0)
    if panel_out:
        for k in range(NP):
            o_ref[k] = acc[:, k * PW:(k + 1) * PW]
    else:
        o_ref[...] = acc


def _sage_combine(s_pm, xd_pm, invc, wl, wr, bl, panel_out):
    grid = (N // M_BLK,)
    if panel_out:
        out_shape = jax.ShapeDtypeStruct((NP, NPAD, PW), jnp.float32)
        out_spec = pl.BlockSpec((NP, M_BLK, PW), lambda m: (0, m, 0))
    else:
        out_shape = jax.ShapeDtypeStruct((N, H), jnp.float32)
        out_spec = pl.BlockSpec((M_BLK, H), lambda m: (m, 0))
    return pl.pallas_call(
        functools.partial(_sage_body, panel_out),
        grid=grid,
        in_specs=[
            pl.BlockSpec((NP, M_BLK, PW), lambda m: (0, m, 0)),
            pl.BlockSpec((NP, M_BLK, PW), lambda m: (0, m, 0)),
            pl.BlockSpec((M_BLK, 1), lambda m: (m, 0)),
            pl.BlockSpec((H, H), lambda m: (0, 0)),
            pl.BlockSpec((H, H), lambda m: (0, 0)),
            pl.BlockSpec((1, H), lambda m: (0, 0)),
        ],
        out_specs=out_spec,
        out_shape=out_shape,
        compiler_params=pltpu.CompilerParams(
            dimension_semantics=("arbitrary",)),
    )(s_pm, xd_pm, invc, wl, wr, bl)


def _proj_body(x_ref, w_ref, b_ref, o_ref):
    acc = jnp.dot(x_ref[...], w_ref[...], preferred_element_type=jnp.float32)
    acc = acc + b_ref[...]
    for k in range(NP):
        o_ref[k] = acc[:, k * PW:(k + 1) * PW]


def _in_proj(x, w, b):
    return pl.pallas_call(
        _proj_body,
        grid=(N // M_BLK,),
        in_specs=[
            pl.BlockSpec((M_BLK, 9), lambda m: (m, 0)),
            pl.BlockSpec((9, H), lambda m: (0, 0)),
            pl.BlockSpec((1, H), lambda m: (0, 0)),
        ],
        out_specs=pl.BlockSpec((NP, M_BLK, PW), lambda m: (0, m, 0)),
        out_shape=jax.ShapeDtypeStruct((NP, NPAD, PW), jnp.float32),
        compiler_params=pltpu.CompilerParams(
            dimension_semantics=("arbitrary",)),
    )(x, w, b)


def _pad_edges(edge):
    npad = EPAD - E
    pad_src = (jnp.arange(npad, dtype=jnp.int32)) % N
    pad_dst = N + (jnp.arange(npad, dtype=jnp.int32) % (NPAD - N))
    src = jnp.concatenate([edge[0], pad_src]).reshape(IDXROWS, 128)
    dst = jnp.concatenate([edge[1], pad_dst]).reshape(IDXROWS, 128)
    return src, dst


def kernel(x_recipe, x_user, edge_ur, edge_ru, emb_user, W_in, b_in,
           Wl0_ur, bl0_ur, Wr0_ur, Wl0_ru, bl0_ru, Wr0_ru,
           Wl1_ur, bl1_ur, Wr1_ur, Wl1_ru, bl1_ru, Wr1_ru):
    src_ur, dst_ur = _pad_edges(edge_ur)
    src_ru, dst_ru = _pad_edges(edge_ru)
    xu_pad = N + (jnp.arange(NPAD - N, dtype=jnp.int32) % (NPAD - N))
    xu_idx = jnp.concatenate(
        [x_user, xu_pad % 100000]).reshape(IDXROWS_PER_SUB, 128)
    zeros = jnp.zeros((ROWS_PER_SUB, PW), jnp.float32)

    ones = jnp.ones((128, PW), jnp.float32)
    invc = _inv_counts(dst_ur, dst_ru, zeros, ones)
    invc_r = invc[0:N, 0:1]
    invc_u = invc[NPAD:NPAD + N, 0:1]

    h_r = _in_proj(x_recipe, W_in, b_in.reshape(1, H))
    h_u = _emb_gather(emb_user, xu_idx)

    s_r = _segsum(h_u, src_ur, dst_ur, zeros)
    s_u = _segsum(h_r, src_ru, dst_ru, zeros)
    h_r1 = _sage_combine(s_r, h_r, invc_r, Wl0_ur, Wr0_ur,
                         bl0_ur.reshape(1, H), True)
    h_u1 = _sage_combine(s_u, h_u, invc_u, Wl0_ru, Wr0_ru,
                         bl0_ru.reshape(1, H), True)

    s_r1 = _segsum(h_u1, src_ur, dst_ur, zeros)
    s_u1 = _segsum(h_r1, src_ru, dst_ru, zeros)
    h_r2 = _sage_combine(s_r1, h_r1, invc_r, Wl1_ur, Wr1_ur,
                         bl1_ur.reshape(1, H), False)
    h_u2 = _sage_combine(s_u1, h_u1, invc_u, Wl1_ru, Wr1_ru,
                         bl1_ru.reshape(1, H), False)
    return (h_u2, h_r2)


# speedup vs baseline: 4.9587x; 1.2419x over previous
import functools

import jax
import jax.numpy as jnp
from jax import lax
from jax.experimental import pallas as pl
from jax.experimental.pallas import tpu as pltpu
from jax.experimental.pallas import tpu_sc as plsc

N = 10000
H = 512
NP = 4
PW = 128
E = 160000
EPAD = 163840
NPAD = 10240
ROWS_PER_SUB = NPAD // 16
IDXROWS = EPAD // 128
IDXROWS_PER_SUB = IDXROWS // 16

_MESH = plsc.VectorSubcoreMesh(core_axis_name="c", subcore_axis_name="s")


def _emb_gather_body(emb, xu_idx, out, idx_v, rows_v):
    c = lax.axis_index("c")
    s = lax.axis_index("s")
    w = s * 2 + c
    for k in range(3):
        b = w + 32 * k

        @pl.when(b < IDXROWS_PER_SUB)
        def _():
            pltpu.sync_copy(xu_idx.at[pl.ds(b, 1)], idx_v)
            pltpu.sync_copy(emb.at[idx_v.at[0]], rows_v)
            off = pl.multiple_of(b * 128, 128)
            for kk in range(NP):
                pltpu.sync_copy(rows_v.at[:, pl.ds(kk * PW, PW)],
                                out.at[kk].at[pl.ds(off, 128), :])


def _emb_gather(emb, xu_idx):
    fn = pl.kernel(
        _emb_gather_body,
        out_type=jax.ShapeDtypeStruct((NP, NPAD, PW), jnp.float32),
        mesh=_MESH,
        scratch_types=[
            pltpu.VMEM((1, 128), jnp.int32),
            pltpu.VMEM((128, H), jnp.float32),
        ],
    )
    return fn(emb, xu_idx)


def _invcnt_body(dst_ur, dst_ru, zeros, ones, out, acc, didx_v, ones_v):
    c = lax.axis_index("c")
    s = lax.axis_index("s")
    base = pl.multiple_of(s * ROWS_PER_SUB, 8)
    pltpu.sync_copy(zeros, acc.at[pl.ds(base, ROWS_PER_SUB)])
    pltpu.sync_copy(ones, ones_v)
    plsc.subcore_barrier()

    def count(dst_ref):
        def blk(i, _):
            r0 = pl.multiple_of(s * IDXROWS_PER_SUB + i * 8, 8)
            pltpu.sync_copy(dst_ref.at[pl.ds(r0, 8)], didx_v)
            for j in range(8):
                pltpu.sync_copy(ones_v, acc.at[didx_v.at[j]], add=True)
            return 0

        lax.fori_loop(0, IDXROWS_PER_SUB // 8, blk, 0)

    @pl.when(c == 0)
    def _():
        count(dst_ur)

    @pl.when(c == 1)
    def _():
        count(dst_ru)

    plsc.subcore_barrier()
    off = pl.multiple_of(c * NPAD + s * ROWS_PER_SUB, 8)
    pltpu.sync_copy(acc.at[pl.ds(base, ROWS_PER_SUB)],
                    out.at[pl.ds(off, ROWS_PER_SUB), :])


def _inv_counts(dst_ur, dst_ru, zeros, ones):
    fn = pl.kernel(
        _invcnt_body,
        out_type=jax.ShapeDtypeStruct((2 * NPAD, PW), jnp.float32),
        mesh=_MESH,
        scratch_types=[
            pltpu.VMEM_SHARED((NPAD, PW), jnp.float32),
            pltpu.VMEM((8, 128), jnp.int32),
            pltpu.VMEM((128, PW), jnp.float32),
        ],
    )
    return fn(dst_ur, dst_ru, zeros, ones)


def _segsum_body(xsrc, src_idx, dst_idx, zeros, out, acc, sidx_v, didx_v,
                 bufs, gsems, ssems):
    c = lax.axis_index("c")
    s = lax.axis_index("s")
    base = pl.multiple_of(s * ROWS_PER_SUB, 8)
    nblk = IDXROWS_PER_SUB
    r0 = pl.multiple_of(s * IDXROWS_PER_SUB, 8)

    def gather(b, k, p):
        pltpu.async_copy(xsrc.at[p].at[sidx_v.at[b]], bufs.at[k],
                         gsems.at[k])

    def gwait(k, p):
        pltpu.make_async_copy(xsrc.at[p].at[sidx_v.at[0]], bufs.at[k],
                              gsems.at[k]).wait()

    def scatter(b, k):
        pltpu.async_copy(bufs.at[k], acc.at[didx_v.at[b]],
                         ssems.at[k], add=True)

    def swait(k):
        pltpu.make_async_copy(bufs.at[k], acc.at[didx_v.at[0]],
                              ssems.at[k]).wait()

    for p in range(NP):

        @pl.when((p // 2) == c)
        def _():
            pltpu.sync_copy(zeros, acc.at[pl.ds(base, ROWS_PER_SUB)])
            plsc.subcore_barrier()

            def grp(i, _):
                @pl.when(i > 0)
                def _():
                    swait(0)
                    swait(1)

                g0 = pl.multiple_of(r0 + i * 8, 8)
                pltpu.sync_copy(src_idx.at[pl.ds(g0, 8)], sidx_v)
                pltpu.sync_copy(dst_idx.at[pl.ds(g0, 8)], didx_v)
                for j in range(8):
                    k = j % 2
                    if j >= 2:
                        swait(k)
                    gather(j, k, p)
                    gwait(k, p)
                    scatter(j, k)
                return 0

            lax.fori_loop(0, nblk // 8, grp, 0)
            swait(0)
            swait(1)
            plsc.subcore_barrier()
            pltpu.sync_copy(acc.at[pl.ds(base, ROWS_PER_SUB)],
                            out.at[p].at[pl.ds(base, ROWS_PER_SUB), :])


def _segsum(xsrc_pm, src_idx, dst_idx, zeros):
    fn = pl.kernel(
        _segsum_body,
        out_type=jax.ShapeDtypeStruct((NP, NPAD, PW), jnp.float32),
        mesh=_MESH,
        scratch_types=[
            pltpu.VMEM_SHARED((NPAD, PW), jnp.float32),
            pltpu.VMEM((8, 128), jnp.int32),
            pltpu.VMEM((8, 128), jnp.int32),
            pltpu.VMEM((2, 128, PW), jnp.float32),
            pltpu.SemaphoreType.DMA((2,)),
            pltpu.SemaphoreType.DMA((2,)),
        ],
    )
    return fn(xsrc_pm, src_idx, dst_idx, zeros)


M_BLK = 400


def _sage_body(panel_out, s_ref, xd_ref, ic_ref, wl_ref, wr_ref, bl_ref,
               o_ref):
    ic = 1.0 / jnp.maximum(ic_ref[...], 1.0)
    acc = jnp.broadcast_to(bl_ref[...], (M_BLK, H)).astype(jnp.float32)
    for k in range(NP):
        wl_k = wl_ref[pl.ds(k * PW, PW), :]
        wr_k = wr_ref[pl.ds(k * PW, PW), :]
        acc = acc + jnp.dot(s_ref[k] * ic, wl_k,
                            preferred_element_type=jnp.float32)
        acc = acc + jnp.dot(xd_ref[k], wr_k,
                            preferred_element_type=jnp.float32)
    acc = jnp.maximum(acc, 0.0)
    if panel_out:
        for k in range(NP):
            o_ref[k] = acc[:, k * PW:(k + 1) * PW]
    else:
        o_ref[...] = acc


def _sage_combine(s_pm, xd_pm, invc, wl, wr, bl, panel_out):
    grid = (N // M_BLK,)
    if panel_out:
        out_shape = jax.ShapeDtypeStruct((NP, NPAD, PW), jnp.float32)
        out_spec = pl.BlockSpec((NP, M_BLK, PW), lambda m: (0, m, 0))
    else:
        out_shape = jax.ShapeDtypeStruct((N, H), jnp.float32)
        out_spec = pl.BlockSpec((M_BLK, H), lambda m: (m, 0))
    return pl.pallas_call(
        functools.partial(_sage_body, panel_out),
        grid=grid,
        in_specs=[
            pl.BlockSpec((NP, M_BLK, PW), lambda m: (0, m, 0)),
            pl.BlockSpec((NP, M_BLK, PW), lambda m: (0, m, 0)),
            pl.BlockSpec((M_BLK, 1), lambda m: (m, 0)),
            pl.BlockSpec((H, H), lambda m: (0, 0)),
            pl.BlockSpec((H, H), lambda m: (0, 0)),
            pl.BlockSpec((1, H), lambda m: (0, 0)),
        ],
        out_specs=out_spec,
        out_shape=out_shape,
        compiler_params=pltpu.CompilerParams(
            dimension_semantics=("arbitrary",)),
    )(s_pm, xd_pm, invc, wl, wr, bl)


def _proj_body(x_ref, w_ref, b_ref, o_ref):
    acc = jnp.dot(x_ref[...], w_ref[...], preferred_element_type=jnp.float32)
    acc = acc + b_ref[...]
    for k in range(NP):
        o_ref[k] = acc[:, k * PW:(k + 1) * PW]


def _in_proj(x, w, b):
    return pl.pallas_call(
        _proj_body,
        grid=(N // M_BLK,),
        in_specs=[
            pl.BlockSpec((M_BLK, 9), lambda m: (m, 0)),
            pl.BlockSpec((9, H), lambda m: (0, 0)),
            pl.BlockSpec((1, H), lambda m: (0, 0)),
        ],
        out_specs=pl.BlockSpec((NP, M_BLK, PW), lambda m: (0, m, 0)),
        out_shape=jax.ShapeDtypeStruct((NP, NPAD, PW), jnp.float32),
        compiler_params=pltpu.CompilerParams(
            dimension_semantics=("arbitrary",)),
    )(x, w, b)


def _pad_edges(edge):
    npad = EPAD - E
    pad_src = (jnp.arange(npad, dtype=jnp.int32)) % N
    pad_dst = N + (jnp.arange(npad, dtype=jnp.int32) % (NPAD - N))
    src = jnp.concatenate([edge[0], pad_src]).reshape(IDXROWS, 128)
    dst = jnp.concatenate([edge[1], pad_dst]).reshape(IDXROWS, 128)
    return src, dst


def kernel(x_recipe, x_user, edge_ur, edge_ru, emb_user, W_in, b_in,
           Wl0_ur, bl0_ur, Wr0_ur, Wl0_ru, bl0_ru, Wr0_ru,
           Wl1_ur, bl1_ur, Wr1_ur, Wl1_ru, bl1_ru, Wr1_ru):
    src_ur, dst_ur = _pad_edges(edge_ur)
    src_ru, dst_ru = _pad_edges(edge_ru)
    xu_pad = N + (jnp.arange(NPAD - N, dtype=jnp.int32) % (NPAD - N))
    xu_idx = jnp.concatenate(
        [x_user, xu_pad % 100000]).reshape(IDXROWS_PER_SUB, 128)
    zeros = jnp.zeros((ROWS_PER_SUB, PW), jnp.float32)

    ones = jnp.ones((128, PW), jnp.float32)
    invc = _inv_counts(dst_ur, dst_ru, zeros, ones)
    invc_r = invc[0:N, 0:1]
    invc_u = invc[NPAD:NPAD + N, 0:1]

    h_r = _in_proj(x_recipe, W_in, b_in.reshape(1, H))
    h_u = _emb_gather(emb_user, xu_idx)

    s_r = _segsum(h_u, src_ur, dst_ur, zeros)
    s_u = _segsum(h_r, src_ru, dst_ru, zeros)
    h_r1 = _sage_combine(s_r, h_r, invc_r, Wl0_ur, Wr0_ur,
                         bl0_ur.reshape(1, H), True)
    h_u1 = _sage_combine(s_u, h_u, invc_u, Wl0_ru, Wr0_ru,
                         bl0_ru.reshape(1, H), True)

    s_r1 = _segsum(h_u1, src_ur, dst_ur, zeros)
    s_u1 = _segsum(h_r1, src_ru, dst_ru, zeros)
    h_r2 = _sage_combine(s_r1, h_r1, invc_r, Wl1_ur, Wr1_ur,
                         bl1_ur.reshape(1, H), False)
    h_u2 = _sage_combine(s_u1, h_u1, invc_u, Wl1_ru, Wr1_ru,
                         bl1_ru.reshape(1, H), False)
    return (h_u2, h_r2)


# speedup vs baseline: 6.4337x; 1.2975x over previous
import functools

import jax
import jax.numpy as jnp
from jax import lax
from jax.experimental import pallas as pl
from jax.experimental.pallas import tpu as pltpu
from jax.experimental.pallas import tpu_sc as plsc

N = 10000
H = 512
NP = 4
PW = 128
E = 160000
EPAD = 163840
EB = 80
EIDXROWS = EPAD // EB
BLKS_PER_SUB = EPAD // 16 // EB
NPAD = 10240
ROWS_PER_SUB = NPAD // 16
IDXROWS_PER_SUB = NPAD // 128

_MESH = plsc.VectorSubcoreMesh(core_axis_name="c", subcore_axis_name="s")


def _emb_gather_body(emb, xu_idx, out, idx_v, rows_v):
    c = lax.axis_index("c")
    s = lax.axis_index("s")
    w = s * 2 + c
    for k in range(3):
        b = w + 32 * k

        @pl.when(b < IDXROWS_PER_SUB)
        def _():
            pltpu.sync_copy(xu_idx.at[pl.ds(b, 1)], idx_v)
            pltpu.sync_copy(emb.at[idx_v.at[0]], rows_v)
            off = pl.multiple_of(b * 128, 128)
            for kk in range(NP):
                pltpu.sync_copy(rows_v.at[:, pl.ds(kk * PW, PW)],
                                out.at[kk].at[pl.ds(off, 128), :])


def _emb_gather(emb, xu_idx):
    fn = pl.kernel(
        _emb_gather_body,
        out_type=jax.ShapeDtypeStruct((NP, NPAD, PW), jnp.float32),
        mesh=_MESH,
        scratch_types=[
            pltpu.VMEM((1, 128), jnp.int32),
            pltpu.VMEM((128, H), jnp.float32),
        ],
    )
    return fn(emb, xu_idx)


def _invcnt_body(e_ur, e_ru, zeros, ones, out, acc, didx_v, ones_v):
    c = lax.axis_index("c")
    s = lax.axis_index("s")
    base = pl.multiple_of(s * ROWS_PER_SUB, 8)
    pltpu.sync_copy(zeros, acc.at[pl.ds(base, ROWS_PER_SUB)])
    pltpu.sync_copy(ones, ones_v)
    plsc.subcore_barrier()

    def count(eref):
        def blk(i, _):
            r0 = pl.multiple_of(s * BLKS_PER_SUB + i * 8, 8)
            pltpu.sync_copy(eref.at[pl.ds(r0, 8)], didx_v)
            for j in range(8):
                pltpu.sync_copy(ones_v, acc.at[didx_v.at[j, 1]], add=True)
            return 0

        lax.fori_loop(0, BLKS_PER_SUB // 8, blk, 0)

    @pl.when(c == 0)
    def _():
        count(e_ur)

    @pl.when(c == 1)
    def _():
        count(e_ru)

    plsc.subcore_barrier()
    off = pl.multiple_of(c * NPAD + s * ROWS_PER_SUB, 8)
    pltpu.sync_copy(acc.at[pl.ds(base, ROWS_PER_SUB)],
                    out.at[pl.ds(off, ROWS_PER_SUB), :])


def _inv_counts(e_ur, e_ru, zeros, ones):
    fn = pl.kernel(
        _invcnt_body,
        out_type=jax.ShapeDtypeStruct((2 * NPAD, PW), jnp.float32),
        mesh=_MESH,
        scratch_types=[
            pltpu.VMEM_SHARED((NPAD, PW), jnp.float32),
            pltpu.VMEM((8, 2, EB), jnp.int32),
            pltpu.VMEM((EB, PW), jnp.float32),
        ],
    )
    return fn(e_ur, e_ru, zeros, ones)


def _segsum_body(xsrc, eidx, zeros, out, acc, idx_v, bufs, gsems, ssems,
                 isems):
    c = lax.axis_index("c")
    s = lax.axis_index("s")
    base = pl.multiple_of(s * ROWS_PER_SUB, 8)
    rbase = s * BLKS_PER_SUB

    def iload(g, sl):
        r = pl.multiple_of(rbase + g * 16, 8)
        pltpu.async_copy(eidx.at[pl.ds(r, 16)], idx_v.at[sl], isems.at[sl])

    def iwait(sl):
        pltpu.make_async_copy(eidx.at[pl.ds(0, 16)], idx_v.at[sl],
                              isems.at[sl]).wait()

    def gather(sl, row, k, p):
        pltpu.async_copy(xsrc.at[p].at[idx_v.at[sl, row, 0]], bufs.at[k],
                         gsems.at[k])

    def gwait(k, p):
        pltpu.make_async_copy(xsrc.at[p].at[idx_v.at[0, 0, 0]], bufs.at[k],
                              gsems.at[k]).wait()

    def scatter(sl, row, k):
        pltpu.async_copy(bufs.at[k], acc.at[idx_v.at[sl, row, 1]],
                         ssems.at[k], add=True)

    def swait(k):
        pltpu.make_async_copy(bufs.at[k], acc.at[idx_v.at[0, 0, 1]],
                              ssems.at[k]).wait()

    for p in range(NP):

        @pl.when((p // 2) == c)
        def _():
            pltpu.sync_copy(zeros, acc.at[pl.ds(base, ROWS_PER_SUB)])
            pltpu.sync_copy(eidx.at[pl.ds(pl.multiple_of(rbase, 8), 16)],
                            idx_v.at[0])
            plsc.subcore_barrier()
            gather(0, 0, 0, p)
            gather(0, 1, 1, p)

            def pair(t, _):
                for gp in range(2):
                    for j in range(16):
                        kc = j % 4
                        kp = (j + 2) % 4
                        if gp == 0 and j < 2:
                            @pl.when(t > 0)
                            def _():
                                swait(kp)
                        else:
                            swait(kp)
                        if j == 1:
                            if gp == 0:
                                iload(2 * t + 1, 1)
                            else:
                                @pl.when(t < 3)
                                def _():
                                    iload(2 * t + 2, 0)
                        if j == 14:
                            if gp == 0:
                                iwait(1)
                            else:
                                @pl.when(t < 3)
                                def _():
                                    iwait(0)
                        if j < 14:
                            gather(gp, j + 2, kp, p)
                        elif gp == 0:
                            gather(1, j - 14, kp, p)
                        else:
                            @pl.when(t < 3)
                            def _():
                                gather(0, j - 14, kp, p)
                        gwait(kc, p)
                        scatter(gp, j, kc)
                return 0

            lax.fori_loop(0, 4, pair, 0)
            swait(2)
            swait(3)
            plsc.subcore_barrier()
            pltpu.sync_copy(acc.at[pl.ds(base, ROWS_PER_SUB)],
                            out.at[p].at[pl.ds(base, ROWS_PER_SUB), :])


def _segsum(xsrc_pm, eidx, zeros):
    fn = pl.kernel(
        _segsum_body,
        out_type=jax.ShapeDtypeStruct((NP, NPAD, PW), jnp.float32),
        mesh=_MESH,
        scratch_types=[
            pltpu.VMEM_SHARED((NPAD, PW), jnp.float32),
            pltpu.VMEM((2, 16, 2, EB), jnp.int32),
            pltpu.VMEM((4, EB, PW), jnp.float32),
            pltpu.SemaphoreType.DMA((4,)),
            pltpu.SemaphoreType.DMA((4,)),
            pltpu.SemaphoreType.DMA((2,)),
        ],
    )
    return fn(xsrc_pm, eidx, zeros)


M_BLK = 400


def _sage_body(panel_out, s_ref, xd_ref, ic_ref, wl_ref, wr_ref, bl_ref,
               o_ref):
    ic = 1.0 / jnp.maximum(ic_ref[...], 1.0)
    acc = jnp.broadcast_to(bl_ref[...], (M_BLK, H)).astype(jnp.float32)
    for k in range(NP):
        wl_k = wl_ref[pl.ds(k * PW, PW), :]
        wr_k = wr_ref[pl.ds(k * PW, PW), :]
        acc = acc + jnp.dot(s_ref[k] * ic, wl_k,
                            preferred_element_type=jnp.float32)
        acc = acc + jnp.dot(xd_ref[k], wr_k,
                            preferred_element_type=jnp.float32)
    acc = jnp.maximum(acc, 0.0)
    if panel_out:
        for k in range(NP):
            o_ref[k] = acc[:, k * PW:(k + 1) * PW]
    else:
        o_ref[...] = acc


def _sage_combine(s_pm, xd_pm, invc, wl, wr, bl, panel_out):
    grid = (N // M_BLK,)
    if panel_out:
        out_shape = jax.ShapeDtypeStruct((NP, NPAD, PW), jnp.float32)
        out_spec = pl.BlockSpec((NP, M_BLK, PW), lambda m: (0, m, 0))
    else:
        out_shape = jax.ShapeDtypeStruct((N, H), jnp.float32)
        out_spec = pl.BlockSpec((M_BLK, H), lambda m: (m, 0))
    return pl.pallas_call(
        functools.partial(_sage_body, panel_out),
        grid=grid,
        in_specs=[
            pl.BlockSpec((NP, M_BLK, PW), lambda m: (0, m, 0)),
            pl.BlockSpec((NP, M_BLK, PW), lambda m: (0, m, 0)),
            pl.BlockSpec((M_BLK, 1), lambda m: (m, 0)),
            pl.BlockSpec((H, H), lambda m: (0, 0)),
            pl.BlockSpec((H, H), lambda m: (0, 0)),
            pl.BlockSpec((1, H), lambda m: (0, 0)),
        ],
        out_specs=out_spec,
        out_shape=out_shape,
        compiler_params=pltpu.CompilerParams(
            dimension_semantics=("arbitrary",)),
    )(s_pm, xd_pm, invc, wl, wr, bl)


def _proj_body(x_ref, w_ref, b_ref, o_ref):
    acc = jnp.dot(x_ref[...], w_ref[...], preferred_element_type=jnp.float32)
    acc = acc + b_ref[...]
    for k in range(NP):
        o_ref[k] = acc[:, k * PW:(k + 1) * PW]


def _in_proj(x, w, b):
    return pl.pallas_call(
        _proj_body,
        grid=(N // M_BLK,),
        in_specs=[
            pl.BlockSpec((M_BLK, 9), lambda m: (m, 0)),
            pl.BlockSpec((9, H), lambda m: (0, 0)),
            pl.BlockSpec((1, H), lambda m: (0, 0)),
        ],
        out_specs=pl.BlockSpec((NP, M_BLK, PW), lambda m: (0, m, 0)),
        out_shape=jax.ShapeDtypeStruct((NP, NPAD, PW), jnp.float32),
        compiler_params=pltpu.CompilerParams(
            dimension_semantics=("arbitrary",)),
    )(x, w, b)


def _pad_edges(edge):
    npad = EPAD - E
    pad_src = (jnp.arange(npad, dtype=jnp.int32)) % N
    pad_dst = N + (jnp.arange(npad, dtype=jnp.int32) % (NPAD - N))
    src = jnp.concatenate([edge[0], pad_src]).reshape(EIDXROWS, EB)
    dst = jnp.concatenate([edge[1], pad_dst]).reshape(EIDXROWS, EB)
    return jnp.stack([src, dst], axis=1)


def kernel(x_recipe, x_user, edge_ur, edge_ru, emb_user, W_in, b_in,
           Wl0_ur, bl0_ur, Wr0_ur, Wl0_ru, bl0_ru, Wr0_ru,
           Wl1_ur, bl1_ur, Wr1_ur, Wl1_ru, bl1_ru, Wr1_ru):
    e_ur = _pad_edges(edge_ur)
    e_ru = _pad_edges(edge_ru)
    xu_pad = N + (jnp.arange(NPAD - N, dtype=jnp.int32) % (NPAD - N))
    xu_idx = jnp.concatenate(
        [x_user, xu_pad % 100000]).reshape(IDXROWS_PER_SUB, 128)
    zeros = jnp.zeros((ROWS_PER_SUB, PW), jnp.float32)

    ones = jnp.ones((EB, PW), jnp.float32)
    invc = _inv_counts(e_ur, e_ru, zeros, ones)
    invc_r = invc[0:N, 0:1]
    invc_u = invc[NPAD:NPAD + N, 0:1]

    h_r = _in_proj(x_recipe, W_in, b_in.reshape(1, H))
    h_u = _emb_gather(emb_user, xu_idx)

    s_r = _segsum(h_u, e_ur, zeros)
    s_u = _segsum(h_r, e_ru, zeros)
    h_r1 = _sage_combine(s_r, h_r, invc_r, Wl0_ur, Wr0_ur,
                         bl0_ur.reshape(1, H), True)
    h_u1 = _sage_combine(s_u, h_u, invc_u, Wl0_ru, Wr0_ru,
                         bl0_ru.reshape(1, H), True)

    s_r1 = _segsum(h_u1, e_ur, zeros)
    s_u1 = _segsum(h_r1, e_ru, zeros)
    h_r2 = _sage_combine(s_r1, h_r1, invc_r, Wl1_ur, Wr1_ur,
                         bl1_ur.reshape(1, H), False)
    h_u2 = _sage_combine(s_u1, h_u1, invc_u, Wl1_ru, Wr1_ru,
                         bl1_ru.reshape(1, H), False)
    return (h_u2, h_r2)


# speedup vs baseline: 6.4731x; 1.0061x over previous
import functools

import jax
import jax.numpy as jnp
from jax import lax
from jax.experimental import pallas as pl
from jax.experimental.pallas import tpu as pltpu
from jax.experimental.pallas import tpu_sc as plsc

N = 10000
H = 512
NP = 4
PW = 128
E = 160000
EPAD = 163840
EB = 80
EIDXROWS = EPAD // EB
BLKS_PER_SUB = EPAD // 16 // EB
NPAD = 10240
ROWS_PER_SUB = NPAD // 16

_MESH = plsc.VectorSubcoreMesh(core_axis_name="c", subcore_axis_name="s")


GB = 64
GBLOCKS = NPAD // GB


def _emb_gather_body(emb, xu_idx, out, idx5, bufs, gsems, wsems):
    c = lax.axis_index("c")
    s = lax.axis_index("s")
    w = s * 2 + c

    def gather(t, sl):
        pltpu.async_copy(emb.at[idx5.at[t]], bufs.at[sl], gsems.at[sl])

    def gwait(sl):
        pltpu.make_async_copy(emb.at[idx5.at[0]], bufs.at[sl],
                              gsems.at[sl]).wait()

    def writes(t, sl):
        off = pl.multiple_of((w + 32 * t) * GB, 8)
        for kk in range(NP):
            pltpu.async_copy(bufs.at[sl].at[:, pl.ds(kk * PW, PW)],
                             out.at[kk].at[pl.ds(off, GB), :], wsems.at[sl])

    def wwait(sl):
        for kk in range(NP):
            pltpu.make_async_copy(bufs.at[sl].at[:, pl.ds(kk * PW, PW)],
                                  out.at[kk].at[pl.ds(0, GB), :],
                                  wsems.at[sl]).wait()

    for t in range(5):
        pltpu.sync_copy(xu_idx.at[pl.ds(w + 32 * t, 1)],
                        idx5.at[pl.ds(t, 1)])
    gather(0, 0)
    for t in range(5):
        sl = t % 2
        if t + 1 < 5:
            if t + 1 >= 2:
                wwait((t + 1) % 2)
            gather(t + 1, (t + 1) % 2)
        gwait(sl)
        writes(t, sl)
    wwait(0)
    wwait(1)


def _emb_gather(emb, xu_idx):
    fn = pl.kernel(
        _emb_gather_body,
        out_type=jax.ShapeDtypeStruct((NP, NPAD, PW), jnp.float32),
        mesh=_MESH,
        name="sc_emb_gather",
        scratch_types=[
            pltpu.VMEM((5, GB), jnp.int32),
            pltpu.VMEM((2, GB, H), jnp.float32),
            pltpu.SemaphoreType.DMA((2,)),
            pltpu.SemaphoreType.DMA((2,)),
        ],
    )
    return fn(emb, xu_idx)


def _invcnt_body(e_ur, e_ru, zeros, ones, out, acc, didx_v, ones_v, ssem):
    c = lax.axis_index("c")
    s = lax.axis_index("s")
    base = pl.multiple_of(s * ROWS_PER_SUB, 8)
    pltpu.sync_copy(zeros, acc.at[pl.ds(base, ROWS_PER_SUB)])
    pltpu.sync_copy(ones, ones_v)
    plsc.subcore_barrier()

    def count(eref):
        r0 = pl.multiple_of(s * BLKS_PER_SUB, 8)
        pltpu.sync_copy(eref.at[pl.ds(r0, BLKS_PER_SUB)], didx_v)

        def swait8():
            for _ in range(8):
                pltpu.make_async_copy(ones_v, acc.at[didx_v.at[0, 1]],
                                      ssem).wait()

        def blk(i, _):
            for j in range(8):
                pltpu.async_copy(ones_v, acc.at[didx_v.at[i * 8 + j, 1]],
                                 ssem, add=True)

            @pl.when(i > 1)
            def _():
                swait8()

            return 0

        lax.fori_loop(0, BLKS_PER_SUB // 8, blk, 0)
        swait8()
        swait8()

    @pl.when(c == 0)
    def _():
        count(e_ur)

    @pl.when(c == 1)
    def _():
        count(e_ru)

    plsc.subcore_barrier()
    off = pl.multiple_of(c * NPAD + s * ROWS_PER_SUB, 8)
    pltpu.sync_copy(acc.at[pl.ds(base, ROWS_PER_SUB)],
                    out.at[pl.ds(off, ROWS_PER_SUB), :])


def _inv_counts(e_ur, e_ru, zeros, ones):
    w = ones.shape[1]
    fn = pl.kernel(
        _invcnt_body,
        out_type=jax.ShapeDtypeStruct((2 * NPAD, w), jnp.float32),
        mesh=_MESH,
        name="sc_edge_counts",
        scratch_types=[
            pltpu.VMEM_SHARED((NPAD, w), jnp.float32),
            pltpu.VMEM((BLKS_PER_SUB, 2, EB), jnp.int32),
            pltpu.VMEM((EB, w), jnp.float32),
            pltpu.SemaphoreType.DMA,
        ],
    )
    return fn(e_ur, e_ru, zeros, ones)


def _segsum_body(xsrc, eidx, zeros, out, acc, idx_v, bufs, gsems, ssems,
                 isems):
    c = lax.axis_index("c")
    s = lax.axis_index("s")
    base = pl.multiple_of(s * ROWS_PER_SUB, 8)
    rbase = s * BLKS_PER_SUB

    def iload(g, sl):
        r = pl.multiple_of(rbase + g * 16, 8)
        pltpu.async_copy(eidx.at[pl.ds(r, 16)], idx_v.at[sl], isems.at[sl])

    def iwait(sl):
        pltpu.make_async_copy(eidx.at[pl.ds(0, 16)], idx_v.at[sl],
                              isems.at[sl]).wait()

    def gather(sl, row, k, p):
        pltpu.async_copy(xsrc.at[p].at[idx_v.at[sl, row, 0]], bufs.at[k],
                         gsems.at[k])

    def gwait(k, p):
        pltpu.make_async_copy(xsrc.at[p].at[idx_v.at[0, 0, 0]], bufs.at[k],
                              gsems.at[k]).wait()

    def scatter(sl, row, k):
        pltpu.async_copy(bufs.at[k], acc.at[idx_v.at[sl, row, 1]],
                         ssems.at[k], add=True)

    def swait(k):
        pltpu.make_async_copy(bufs.at[k], acc.at[idx_v.at[0, 0, 1]],
                              ssems.at[k]).wait()

    for p in range(NP):

        @pl.when((p // 2) == c)
        def _():
            pltpu.sync_copy(zeros, acc.at[pl.ds(base, ROWS_PER_SUB)])
            pltpu.sync_copy(eidx.at[pl.ds(pl.multiple_of(rbase, 8), 16)],
                            idx_v.at[0])
            plsc.subcore_barrier()
            gather(0, 0, 0, p)
            gather(0, 1, 1, p)

            def pair(t, _):
                for gp in range(2):
                    for j in range(16):
                        kc = j % 4
                        kp = (j + 2) % 4
                        if gp == 0 and j < 2:
                            @pl.when(t > 0)
                            def _():
                                swait(kp)
                        else:
                            swait(kp)
                        if j == 1:
                            if gp == 0:
                                iload(2 * t + 1, 1)
                            else:
                                @pl.when(t < 3)
                                def _():
                                    iload(2 * t + 2, 0)
                        if j == 14:
                            if gp == 0:
                                iwait(1)
                            else:
                                @pl.when(t < 3)
                                def _():
                                    iwait(0)
                        if j < 14:
                            gather(gp, j + 2, kp, p)
                        elif gp == 0:
                            gather(1, j - 14, kp, p)
                        else:
                            @pl.when(t < 3)
                            def _():
                                gather(0, j - 14, kp, p)
                        gwait(kc, p)
                        scatter(gp, j, kc)
                return 0

            lax.fori_loop(0, 4, pair, 0)
            swait(2)
            swait(3)
            plsc.subcore_barrier()
            pltpu.sync_copy(acc.at[pl.ds(base, ROWS_PER_SUB)],
                            out.at[p].at[pl.ds(base, ROWS_PER_SUB), :])


def _segsum(xsrc_pm, eidx, zeros):
    fn = pl.kernel(
        _segsum_body,
        out_type=jax.ShapeDtypeStruct((NP, NPAD, PW), jnp.float32),
        mesh=_MESH,
        name="sc_segsum",
        scratch_types=[
            pltpu.VMEM_SHARED((NPAD, PW), jnp.float32),
            pltpu.VMEM((2, 16, 2, EB), jnp.int32),
            pltpu.VMEM((4, EB, PW), jnp.float32),
            pltpu.SemaphoreType.DMA((4,)),
            pltpu.SemaphoreType.DMA((4,)),
            pltpu.SemaphoreType.DMA((2,)),
        ],
    )
    return fn(xsrc_pm, eidx, zeros)


M_BLK = 400


def _sage_body(panel_out, s_ref, xd_ref, ic_ref, wl_ref, wr_ref, bl_ref,
               o_ref):
    ic = 1.0 / jnp.maximum(ic_ref[...], 1.0)
    acc = jnp.broadcast_to(bl_ref[...], (M_BLK, H)).astype(jnp.float32)
    for k in range(NP):
        wl_k = wl_ref[pl.ds(k * PW, PW), :]
        wr_k = wr_ref[pl.ds(k * PW, PW), :]
        acc = acc + jnp.dot(s_ref[k] * ic, wl_k,
                            preferred_element_type=jnp.float32)
        acc = acc + jnp.dot(xd_ref[k], wr_k,
                            preferred_element_type=jnp.float32)
    acc = jnp.maximum(acc, 0.0)
    if panel_out:
        for k in range(NP):
            o_ref[k] = acc[:, k * PW:(k + 1) * PW]
    else:
        o_ref[...] = acc


def _sage_combine(s_pm, xd_pm, invc, wl, wr, bl, panel_out):
    grid = (N // M_BLK,)
    if panel_out:
        out_shape = jax.ShapeDtypeStruct((NP, NPAD, PW), jnp.float32)
        out_spec = pl.BlockSpec((NP, M_BLK, PW), lambda m: (0, m, 0))
    else:
        out_shape = jax.ShapeDtypeStruct((N, H), jnp.float32)
        out_spec = pl.BlockSpec((M_BLK, H), lambda m: (m, 0))
    return pl.pallas_call(
        functools.partial(_sage_body, panel_out),
        grid=grid,
        in_specs=[
            pl.BlockSpec((NP, M_BLK, PW), lambda m: (0, m, 0)),
            pl.BlockSpec((NP, M_BLK, PW), lambda m: (0, m, 0)),
            pl.BlockSpec((M_BLK, 1), lambda m: (m, 0)),
            pl.BlockSpec((H, H), lambda m: (0, 0)),
            pl.BlockSpec((H, H), lambda m: (0, 0)),
            pl.BlockSpec((1, H), lambda m: (0, 0)),
        ],
        out_specs=out_spec,
        out_shape=out_shape,
        compiler_params=pltpu.CompilerParams(
            dimension_semantics=("arbitrary",)),
    )(s_pm, xd_pm, invc, wl, wr, bl)


def _proj_body(x_ref, w_ref, b_ref, o_ref):
    acc = jnp.dot(x_ref[...], w_ref[...], preferred_element_type=jnp.float32)
    acc = acc + b_ref[...]
    for k in range(NP):
        o_ref[k] = acc[:, k * PW:(k + 1) * PW]


def _in_proj(x, w, b):
    return pl.pallas_call(
        _proj_body,
        grid=(N // M_BLK,),
        in_specs=[
            pl.BlockSpec((M_BLK, 9), lambda m: (m, 0)),
            pl.BlockSpec((9, H), lambda m: (0, 0)),
            pl.BlockSpec((1, H), lambda m: (0, 0)),
        ],
        out_specs=pl.BlockSpec((NP, M_BLK, PW), lambda m: (0, m, 0)),
        out_shape=jax.ShapeDtypeStruct((NP, NPAD, PW), jnp.float32),
        compiler_params=pltpu.CompilerParams(
            dimension_semantics=("arbitrary",)),
    )(x, w, b)


def _pad_edges(edge):
    npad = EPAD - E
    pad_src = (jnp.arange(npad, dtype=jnp.int32)) % N
    pad_dst = N + (jnp.arange(npad, dtype=jnp.int32) % (NPAD - N))
    src = jnp.concatenate([edge[0], pad_src]).reshape(EIDXROWS, EB)
    dst = jnp.concatenate([edge[1], pad_dst]).reshape(EIDXROWS, EB)
    return jnp.stack([src, dst], axis=1)


def kernel(x_recipe, x_user, edge_ur, edge_ru, emb_user, W_in, b_in,
           Wl0_ur, bl0_ur, Wr0_ur, Wl0_ru, bl0_ru, Wr0_ru,
           Wl1_ur, bl1_ur, Wr1_ur, Wl1_ru, bl1_ru, Wr1_ru):
    e_ur = _pad_edges(edge_ur)
    e_ru = _pad_edges(edge_ru)
    xu_pad = N + (jnp.arange(NPAD - N, dtype=jnp.int32) % (NPAD - N))
    xu_idx = jnp.concatenate(
        [x_user, xu_pad % 100000]).reshape(GBLOCKS, GB)
    zeros = jnp.zeros((ROWS_PER_SUB, PW), jnp.float32)

    ones = jnp.ones((EB, PW), jnp.float32)
    invc = _inv_counts(e_ur, e_ru, zeros, ones)
    invc_r = invc[0:N, 0:1]
    invc_u = invc[NPAD:NPAD + N, 0:1]

    h_r = _in_proj(x_recipe, W_in, b_in.reshape(1, H))
    h_u = _emb_gather(emb_user, xu_idx)

    s_r = _segsum(h_u, e_ur, zeros)
    s_u = _segsum(h_r, e_ru, zeros)
    h_r1 = _sage_combine(s_r, h_r, invc_r, Wl0_ur, Wr0_ur,
                         bl0_ur.reshape(1, H), True)
    h_u1 = _sage_combine(s_u, h_u, invc_u, Wl0_ru, Wr0_ru,
                         bl0_ru.reshape(1, H), True)

    s_r1 = _segsum(h_u1, e_ur, zeros)
    s_u1 = _segsum(h_r1, e_ru, zeros)
    h_r2 = _sage_combine(s_r1, h_r1, invc_r, Wl1_ur, Wr1_ur,
                         bl1_ur.reshape(1, H), False)
    h_u2 = _sage_combine(s_u1, h_u1, invc_u, Wl1_ru, Wr1_ru,
                         bl1_ru.reshape(1, H), False)
    return (h_u2, h_r2)


# speedup vs baseline: 6.6039x; 1.0202x over previous
import functools

import jax
import jax.numpy as jnp
from jax import lax
from jax.experimental import pallas as pl
from jax.experimental.pallas import tpu as pltpu
from jax.experimental.pallas import tpu_sc as plsc

N = 10000
H = 512
NP = 4
PW = 128
E = 160000
EPAD = 163840
EB = 64
EIDXROWS = EPAD // EB
BLKS_PER_SUB = EPAD // 16 // EB
GL = 10
NPAIR = BLKS_PER_SUB // GL // 2
NPAD = 10240
ROWS_PER_SUB = NPAD // 16

_MESH = plsc.VectorSubcoreMesh(core_axis_name="c", subcore_axis_name="s")


GB = 64
GBLOCKS = NPAD // GB


def _emb_gather_body(emb, xu_idx, out, idx5, bufs, gsems, wsems):
    c = lax.axis_index("c")
    s = lax.axis_index("s")
    w = s * 2 + c

    def gather(t, sl):
        pltpu.async_copy(emb.at[idx5.at[t]], bufs.at[sl], gsems.at[sl])

    def gwait(sl):
        pltpu.make_async_copy(emb.at[idx5.at[0]], bufs.at[sl],
                              gsems.at[sl]).wait()

    def writes(t, sl):
        off = pl.multiple_of((w + 32 * t) * GB, 8)
        for kk in range(NP):
            pltpu.async_copy(bufs.at[sl].at[:, pl.ds(kk * PW, PW)],
                             out.at[kk].at[pl.ds(off, GB), :], wsems.at[sl])

    def wwait(sl):
        for kk in range(NP):
            pltpu.make_async_copy(bufs.at[sl].at[:, pl.ds(kk * PW, PW)],
                                  out.at[kk].at[pl.ds(0, GB), :],
                                  wsems.at[sl]).wait()

    for t in range(5):
        pltpu.sync_copy(xu_idx.at[pl.ds(w + 32 * t, 1)],
                        idx5.at[pl.ds(t, 1)])
    gather(0, 0)
    for t in range(5):
        sl = t % 2
        if t + 1 < 5:
            if t + 1 >= 2:
                wwait((t + 1) % 2)
            gather(t + 1, (t + 1) % 2)
        gwait(sl)
        writes(t, sl)
    wwait(0)
    wwait(1)


def _emb_gather(emb, xu_idx):
    fn = pl.kernel(
        _emb_gather_body,
        out_type=jax.ShapeDtypeStruct((NP, NPAD, PW), jnp.float32),
        mesh=_MESH,
        name="sc_emb_gather",
        scratch_types=[
            pltpu.VMEM((5, GB), jnp.int32),
            pltpu.VMEM((2, GB, H), jnp.float32),
            pltpu.SemaphoreType.DMA((2,)),
            pltpu.SemaphoreType.DMA((2,)),
        ],
    )
    return fn(emb, xu_idx)


def _invcnt_body(e_ur, e_ru, zeros, ones, out, acc, didx_v, ones_v, ssem):
    c = lax.axis_index("c")
    s = lax.axis_index("s")
    base = pl.multiple_of(s * ROWS_PER_SUB, 8)
    pltpu.sync_copy(zeros, acc.at[pl.ds(base, ROWS_PER_SUB)])
    pltpu.sync_copy(ones, ones_v)
    plsc.subcore_barrier()

    def count(eref):
        r0 = pl.multiple_of(s * BLKS_PER_SUB, 8)
        pltpu.sync_copy(eref.at[pl.ds(r0, BLKS_PER_SUB)], didx_v)

        def swait8():
            for _ in range(8):
                pltpu.make_async_copy(ones_v, acc.at[didx_v.at[0, 1]],
                                      ssem).wait()

        def blk(i, _):
            for j in range(8):
                pltpu.async_copy(ones_v, acc.at[didx_v.at[i * 8 + j, 1]],
                                 ssem, add=True)

            @pl.when(i > 1)
            def _():
                swait8()

            return 0

        lax.fori_loop(0, BLKS_PER_SUB // 8, blk, 0)
        swait8()
        swait8()

    @pl.when(c == 0)
    def _():
        count(e_ur)

    @pl.when(c == 1)
    def _():
        count(e_ru)

    plsc.subcore_barrier()
    off = pl.multiple_of(c * NPAD + s * ROWS_PER_SUB, 8)
    pltpu.sync_copy(acc.at[pl.ds(base, ROWS_PER_SUB)],
                    out.at[pl.ds(off, ROWS_PER_SUB), :])


def _inv_counts(e_ur, e_ru, zeros, ones):
    w = ones.shape[1]
    fn = pl.kernel(
        _invcnt_body,
        out_type=jax.ShapeDtypeStruct((2 * NPAD, w), jnp.float32),
        mesh=_MESH,
        name="sc_edge_counts",
        scratch_types=[
            pltpu.VMEM_SHARED((NPAD, w), jnp.float32),
            pltpu.VMEM((BLKS_PER_SUB, 2, EB), jnp.int32),
            pltpu.VMEM((EB, w), jnp.float32),
            pltpu.SemaphoreType.DMA,
        ],
    )
    return fn(e_ur, e_ru, zeros, ones)


def _segsum_body(xsrc, eidx, zeros, out, acc, idx_v, bufs, gsems, ssems,
                 isems):
    c = lax.axis_index("c")
    s = lax.axis_index("s")
    base = pl.multiple_of(s * ROWS_PER_SUB, 8)
    rbase = s * BLKS_PER_SUB

    def iload(g, sl):
        r = pl.multiple_of(rbase + g * GL, 2)
        pltpu.async_copy(eidx.at[pl.ds(r, GL)], idx_v.at[sl], isems.at[sl])

    def iwait(sl):
        pltpu.make_async_copy(eidx.at[pl.ds(0, GL)], idx_v.at[sl],
                              isems.at[sl]).wait()

    def gather(sl, row, k, p):
        pltpu.async_copy(xsrc.at[p].at[idx_v.at[sl, row, 0]], bufs.at[k],
                         gsems.at[k])

    def gwait(k, p):
        pltpu.make_async_copy(xsrc.at[p].at[idx_v.at[0, 0, 0]], bufs.at[k],
                              gsems.at[k]).wait()

    def scatter(sl, row, k):
        pltpu.async_copy(bufs.at[k], acc.at[idx_v.at[sl, row, 1]],
                         ssems.at[k], add=True)

    def swait(k):
        pltpu.make_async_copy(bufs.at[k], acc.at[idx_v.at[0, 0, 1]],
                              ssems.at[k]).wait()

    for p in range(NP):

        @pl.when((p // 2) == c)
        def _():
            pltpu.sync_copy(zeros, acc.at[pl.ds(base, ROWS_PER_SUB)])
            pltpu.sync_copy(eidx.at[pl.ds(pl.multiple_of(rbase, 8), GL)],
                            idx_v.at[0])
            plsc.subcore_barrier()
            gather(0, 0, 0, p)
            gather(0, 1, 1, p)

            def pair(t, _):
                for gp in range(2):
                    for j in range(GL):
                        kc = j % 5
                        kp = (j + 2) % 5
                        if gp == 0 and j < 3:
                            @pl.when(t > 0)
                            def _():
                                swait(kp)
                        else:
                            swait(kp)
                        if j == 3:
                            if gp == 0:
                                iload(2 * t + 1, 1)
                            else:
                                @pl.when(t < NPAIR - 1)
                                def _():
                                    iload(2 * t + 2, 0)
                        if j == GL - 3:
                            if gp == 0:
                                iwait(1)
                            else:
                                @pl.when(t < NPAIR - 1)
                                def _():
                                    iwait(0)
                        if j < GL - 2:
                            gather(gp, j + 2, kp, p)
                        elif gp == 0:
                            gather(1, j - (GL - 2), kp, p)
                        else:
                            @pl.when(t < NPAIR - 1)
                            def _():
                                gather(0, j - (GL - 2), kp, p)
                        gwait(kc, p)
                        scatter(gp, j, kc)
                return 0

            lax.fori_loop(0, NPAIR, pair, 0)
            swait(2)
            swait(3)
            swait(4)
            plsc.subcore_barrier()
            pltpu.sync_copy(acc.at[pl.ds(base, ROWS_PER_SUB)],
                            out.at[p].at[pl.ds(base, ROWS_PER_SUB), :])


def _segsum(xsrc_pm, eidx, zeros):
    fn = pl.kernel(
        _segsum_body,
        out_type=jax.ShapeDtypeStruct((NP, NPAD, PW), jnp.float32),
        mesh=_MESH,
        name="sc_segsum",
        scratch_types=[
            pltpu.VMEM_SHARED((NPAD, PW), jnp.float32),
            pltpu.VMEM((2, GL, 2, EB), jnp.int32),
            pltpu.VMEM((5, EB, PW), jnp.float32),
            pltpu.SemaphoreType.DMA((5,)),
            pltpu.SemaphoreType.DMA((5,)),
            pltpu.SemaphoreType.DMA((2,)),
        ],
    )
    return fn(xsrc_pm, eidx, zeros)


M_BLK = 400


def _sage_body(panel_out, s_ref, xd_ref, ic_ref, wl_ref, wr_ref, bl_ref,
               o_ref):
    ic = 1.0 / jnp.maximum(ic_ref[...], 1.0)
    acc = jnp.broadcast_to(bl_ref[...], (M_BLK, H)).astype(jnp.float32)
    for k in range(NP):
        wl_k = wl_ref[pl.ds(k * PW, PW), :]
        wr_k = wr_ref[pl.ds(k * PW, PW), :]
        acc = acc + jnp.dot(s_ref[k] * ic, wl_k,
                            preferred_element_type=jnp.float32)
        acc = acc + jnp.dot(xd_ref[k], wr_k,
                            preferred_element_type=jnp.float32)
    acc = jnp.maximum(acc, 0.0)
    if panel_out:
        for k in range(NP):
            o_ref[k] = acc[:, k * PW:(k + 1) * PW]
    else:
        o_ref[...] = acc


def _sage_combine(s_pm, xd_pm, invc, wl, wr, bl, panel_out):
    grid = (N // M_BLK,)
    if panel_out:
        out_shape = jax.ShapeDtypeStruct((NP, NPAD, PW), jnp.float32)
        out_spec = pl.BlockSpec((NP, M_BLK, PW), lambda m: (0, m, 0))
    else:
        out_shape = jax.ShapeDtypeStruct((N, H), jnp.float32)
        out_spec = pl.BlockSpec((M_BLK, H), lambda m: (m, 0))
    return pl.pallas_call(
        functools.partial(_sage_body, panel_out),
        grid=grid,
        in_specs=[
            pl.BlockSpec((NP, M_BLK, PW), lambda m: (0, m, 0)),
            pl.BlockSpec((NP, M_BLK, PW), lambda m: (0, m, 0)),
            pl.BlockSpec((M_BLK, 1), lambda m: (m, 0)),
            pl.BlockSpec((H, H), lambda m: (0, 0)),
            pl.BlockSpec((H, H), lambda m: (0, 0)),
            pl.BlockSpec((1, H), lambda m: (0, 0)),
        ],
        out_specs=out_spec,
        out_shape=out_shape,
        compiler_params=pltpu.CompilerParams(
            dimension_semantics=("arbitrary",)),
    )(s_pm, xd_pm, invc, wl, wr, bl)


def _proj_body(x_ref, w_ref, b_ref, o_ref):
    acc = jnp.dot(x_ref[...], w_ref[...], preferred_element_type=jnp.float32)
    acc = acc + b_ref[...]
    for k in range(NP):
        o_ref[k] = acc[:, k * PW:(k + 1) * PW]


def _in_proj(x, w, b):
    return pl.pallas_call(
        _proj_body,
        grid=(N // M_BLK,),
        in_specs=[
            pl.BlockSpec((M_BLK, 9), lambda m: (m, 0)),
            pl.BlockSpec((9, H), lambda m: (0, 0)),
            pl.BlockSpec((1, H), lambda m: (0, 0)),
        ],
        out_specs=pl.BlockSpec((NP, M_BLK, PW), lambda m: (0, m, 0)),
        out_shape=jax.ShapeDtypeStruct((NP, NPAD, PW), jnp.float32),
        compiler_params=pltpu.CompilerParams(
            dimension_semantics=("arbitrary",)),
    )(x, w, b)


def _pad_edges(edge):
    npad = EPAD - E
    pad_src = (jnp.arange(npad, dtype=jnp.int32)) % N
    pad_dst = N + (jnp.arange(npad, dtype=jnp.int32) % (NPAD - N))
    src = jnp.concatenate([edge[0], pad_src]).reshape(EIDXROWS, EB)
    dst = jnp.concatenate([edge[1], pad_dst]).reshape(EIDXROWS, EB)
    return jnp.stack([src, dst], axis=1)


def kernel(x_recipe, x_user, edge_ur, edge_ru, emb_user, W_in, b_in,
           Wl0_ur, bl0_ur, Wr0_ur, Wl0_ru, bl0_ru, Wr0_ru,
           Wl1_ur, bl1_ur, Wr1_ur, Wl1_ru, bl1_ru, Wr1_ru):
    e_ur = _pad_edges(edge_ur)
    e_ru = _pad_edges(edge_ru)
    xu_pad = N + (jnp.arange(NPAD - N, dtype=jnp.int32) % (NPAD - N))
    xu_idx = jnp.concatenate(
        [x_user, xu_pad % 100000]).reshape(GBLOCKS, GB)
    zeros = jnp.zeros((ROWS_PER_SUB, PW), jnp.float32)

    ones = jnp.ones((EB, PW), jnp.float32)
    invc = _inv_counts(e_ur, e_ru, zeros, ones)
    invc_r = invc[0:N, 0:1]
    invc_u = invc[NPAD:NPAD + N, 0:1]

    h_r = _in_proj(x_recipe, W_in, b_in.reshape(1, H))
    h_u = _emb_gather(emb_user, xu_idx)

    s_r = _segsum(h_u, e_ur, zeros)
    s_u = _segsum(h_r, e_ru, zeros)
    h_r1 = _sage_combine(s_r, h_r, invc_r, Wl0_ur, Wr0_ur,
                         bl0_ur.reshape(1, H), True)
    h_u1 = _sage_combine(s_u, h_u, invc_u, Wl0_ru, Wr0_ru,
                         bl0_ru.reshape(1, H), True)

    s_r1 = _segsum(h_u1, e_ur, zeros)
    s_u1 = _segsum(h_r1, e_ru, zeros)
    h_r2 = _sage_combine(s_r1, h_r1, invc_r, Wl1_ur, Wr1_ur,
                         bl1_ur.reshape(1, H), False)
    h_u2 = _sage_combine(s_u1, h_u1, invc_u, Wl1_ru, Wr1_ru,
                         bl1_ru.reshape(1, H), False)
    return (h_u2, h_r2)
